# speedup vs baseline: 1.1877x; 1.1877x over previous
import jax
import jax.numpy as jnp
from jax.experimental import pallas as pl
from jax.experimental.pallas import tpu as pltpu

_N_S = 256
_N_L = 512
_N_T = 128


def _fused_kernel(s_ref, l_ref, t_ref, tab_ref, w2_ref, b2_ref,
                  w3_ref, b3_ref, o_ref):
    tb = s_ref.shape[0]
    f32 = jnp.float32

    cs = jax.lax.broadcasted_iota(jnp.int32, (tb, _N_S), 1)
    cl = jax.lax.broadcasted_iota(jnp.int32, (tb, _N_L), 1)
    ct = jax.lax.broadcasted_iota(jnp.int32, (tb, _N_T), 1)
    hot_s = jnp.where(cs == s_ref[...], f32(1.0), f32(0.0))
    hot_l = jnp.where(cl == l_ref[...], f32(1.0), f32(0.0))
    hot_t = jnp.where(ct == t_ref[...], f32(1.0), f32(0.0))
    hot = jnp.concatenate([hot_s, hot_l, hot_t], axis=1)

    h1 = jnp.dot(hot, tab_ref[...], preferred_element_type=f32)
    h1 = jnp.maximum(h1, f32(0.0))

    h2 = jnp.dot(h1, w2_ref[...], preferred_element_type=f32) + b2_ref[...]
    h2 = jnp.maximum(h2, f32(0.0))

    o_ref[...] = (jnp.dot(h2, w3_ref[...], preferred_element_type=f32)
                  + b3_ref[...])


def kernel(s_idx, l_idx, t_idx, tab, b1, w2, b2, w3, b3):
    B = s_idx.shape[0]
    O = w3.shape[1]
    tb = 4096

    tab2 = tab.at[_N_S + _N_L:_N_S + _N_L + _N_T].add(b1)

    Bp = ((B + tb - 1) // tb) * tb
    grid = (Bp // tb,)

    def prep_idx(idx):
        idx = idx.astype(jnp.int32).reshape(-1, 1)
        if Bp != B:
            idx = jnp.pad(idx, ((0, Bp - B), (0, 0)))
        return idx

    s2, l2, t2 = prep_idx(s_idx), prep_idx(l_idx), prep_idx(t_idx)

    idx_spec = pl.BlockSpec((tb, 1), lambda i: (i, 0))

    def resident(a):
        return pl.BlockSpec(a.shape, lambda i: (0, 0))

    out = pl.pallas_call(
        _fused_kernel,
        out_shape=jax.ShapeDtypeStruct((Bp, O), jnp.float32),
        grid_spec=pltpu.PrefetchScalarGridSpec(
            num_scalar_prefetch=0,
            grid=grid,
            in_specs=[idx_spec, idx_spec, idx_spec,
                      resident(tab2), resident(w2), resident(b2),
                      resident(w3), resident(b3)],
            out_specs=pl.BlockSpec((tb, O), lambda i: (i, 0)),
        ),
        compiler_params=pltpu.CompilerParams(
            dimension_semantics=("parallel",),
            vmem_limit_bytes=64 * 1024 * 1024),
    )(s2, l2, t2, tab2, w2, b2, w3, b3)

    return out[:B] if Bp != B else out


# speedup vs baseline: 1.2024x; 1.0124x over previous
import jax
import jax.numpy as jnp
from jax.experimental import pallas as pl
from jax.experimental.pallas import tpu as pltpu

_N_S = 256
_N_L = 512
_N_T = 128


def _fused_kernel(s_ref, l_ref, t_ref, tab_ref, w2_ref, b2_ref,
                  w3_ref, b3_ref, o_ref):
    tb = s_ref.shape[0]
    f32 = jnp.float32
    bf16 = jnp.bfloat16

    cs = jax.lax.broadcasted_iota(jnp.int32, (tb, _N_S), 1)
    cl = jax.lax.broadcasted_iota(jnp.int32, (tb, _N_L), 1)
    ct = jax.lax.broadcasted_iota(jnp.int32, (tb, 2 * _N_T), 1)
    one = f32(1.0)
    zero = f32(0.0)
    hot_s = jnp.where(cs == s_ref[...], one, zero)
    hot_l = jnp.where(cl == l_ref[...], one, zero)
    hot_t = jnp.where(ct == t_ref[...], one, zero)
    hot = jnp.concatenate([hot_s, hot_l, hot_t], axis=1).astype(bf16)

    h1 = jnp.dot(hot, tab_ref[...], preferred_element_type=f32)
    h1 = jnp.maximum(h1, f32(0.0)).astype(bf16)

    h2 = jnp.dot(h1, w2_ref[...], preferred_element_type=f32) + b2_ref[...]
    h2 = jnp.maximum(h2, f32(0.0)).astype(bf16)

    o_ref[...] = (jnp.dot(h2, w3_ref[...], preferred_element_type=f32)
                  + b3_ref[...])


def kernel(s_idx, l_idx, t_idx, tab, b1, w2, b2, w3, b3):
    B = s_idx.shape[0]
    O = w3.shape[1]
    tb = 8192

    tab2 = tab.at[_N_S + _N_L:_N_S + _N_L + _N_T].add(b1)
    tab2 = jnp.pad(tab2, ((0, 1024 - tab2.shape[0]), (0, 0))).astype(jnp.bfloat16)
    w2b = w2.astype(jnp.bfloat16)
    w3b = w3.astype(jnp.bfloat16)

    Bp = ((B + tb - 1) // tb) * tb
    grid = (Bp // tb,)

    def prep_idx(idx):
        idx = idx.astype(jnp.int32).reshape(-1, 1)
        if Bp != B:
            idx = jnp.pad(idx, ((0, Bp - B), (0, 0)))
        return idx

    s2, l2, t2 = prep_idx(s_idx), prep_idx(l_idx), prep_idx(t_idx)

    idx_spec = pl.BlockSpec((tb, 1), lambda i: (i, 0))

    def resident(a):
        return pl.BlockSpec(a.shape, lambda i: (0, 0))

    out = pl.pallas_call(
        _fused_kernel,
        out_shape=jax.ShapeDtypeStruct((Bp, O), jnp.float32),
        grid_spec=pltpu.PrefetchScalarGridSpec(
            num_scalar_prefetch=0,
            grid=grid,
            in_specs=[idx_spec, idx_spec, idx_spec,
                      resident(tab2), resident(w2b), resident(b2),
                      resident(w3b), resident(b3)],
            out_specs=pl.BlockSpec((tb, O), lambda i: (i, 0)),
        ),
        compiler_params=pltpu.CompilerParams(
            dimension_semantics=("arbitrary",),
            vmem_limit_bytes=64 * 1024 * 1024),
    )(s2, l2, t2, tab2, w2b, b2, w3b, b3)

    return out[:B] if Bp != B else out
